# Initial kernel scaffold; baseline (speedup 1.0000x reference)
import jax
import jax.numpy as jnp
from jax.experimental import pallas as pl


def kernel(x, node_graph_idx, W, b):
    raise NotImplementedError("write your pallas kernel here")


# speedup vs baseline: 1.8032x; 1.8032x over previous
import jax
import jax.numpy as jnp
from jax import lax
from jax.experimental import pallas as pl
from jax.experimental.pallas import tpu as pltpu
from jax.experimental.pallas import tpu_sc as plsc

N = 100000
G = 1024
D = 64

NC = 2
NS = 16
NW = NC * NS

CH = 800
JROWS = 8
JLEN = 100
NCHUNK = N // CH
GPT = G // NS
CW = 16


def _sc_body(x3, idx3, ones_h, z64_h, z1_h, psum, pcnt,
             xbuf, ibuf, obuf, zbufA, zbufC, acc, cacc):
    c = lax.axis_index("c")
    s = lax.axis_index("s")
    w = s * NC + c

    pltpu.sync_copy(ones_h, obuf)
    pltpu.sync_copy(z64_h, zbufA)
    pltpu.sync_copy(z1_h, zbufC)
    pltpu.sync_copy(zbufA, acc.at[pl.ds(s * GPT, GPT)])
    pltpu.sync_copy(zbufC, cacc.at[pl.ds(s * GPT, GPT)])
    plsc.subcore_barrier()

    for k in range((NCHUNK + NW - 1) // NW):
        cid = w + NW * k

        @pl.when(cid < NCHUNK)
        def _():
            pltpu.sync_copy(x3.at[cid], xbuf)
            pltpu.sync_copy(idx3.at[cid], ibuf)
            for j in range(JROWS):
                pltpu.sync_copy(xbuf.at[pl.ds(j * JLEN, JLEN)],
                                acc.at[ibuf.at[j]], add=True)
                pltpu.sync_copy(obuf.at[pl.ds(j * JLEN, JLEN)],
                                cacc.at[ibuf.at[j]], add=True)

    plsc.subcore_barrier()

    pltpu.sync_copy(acc.at[pl.ds(s * GPT, GPT)], zbufA)
    pltpu.sync_copy(zbufA, psum.at[c, pl.ds(s * GPT, GPT)])
    pltpu.sync_copy(cacc.at[pl.ds(s * GPT, GPT)], zbufC)
    pltpu.sync_copy(zbufC, pcnt.at[c, pl.ds(s * GPT, GPT)])


def _make_sc_call():
    mesh = plsc.VectorSubcoreMesh(core_axis_name="c", subcore_axis_name="s",
                                  num_cores=NC, num_subcores=NS)

    return pl.kernel(
        _sc_body,
        out_type=(
            jax.ShapeDtypeStruct((NC, G, D), jnp.float32),
            jax.ShapeDtypeStruct((NC, G, CW), jnp.float32),
        ),
        mesh=mesh,
        compiler_params=pltpu.CompilerParams(use_tc_tiling_on_sc=False),
        scratch_types=[
            pltpu.VMEM((CH, D), jnp.float32),
            pltpu.VMEM((JROWS, JLEN), jnp.int32),
            pltpu.VMEM((CH, CW), jnp.float32),
            pltpu.VMEM((GPT, D), jnp.float32),
            pltpu.VMEM((GPT, CW), jnp.float32),
            pltpu.VMEM_SHARED((G, D), jnp.float32),
            pltpu.VMEM_SHARED((G, CW), jnp.float32),
        ],
    )


def _tc_body(ps_ref, pc_ref, w_ref, b_ref, out_ref):
    ps = ps_ref[...]
    pc = pc_ref[...]
    seg = ps[0] + ps[1]
    cnt = (pc[0] + pc[1])[:, 0:1]
    dot = jnp.dot(seg, w_ref[...], preferred_element_type=jnp.float32)
    out_ref[...] = (dot + cnt * b_ref[...]) / jnp.maximum(cnt, 1.0)


def kernel(x, node_graph_idx, W, b):
    x3 = x.reshape(NCHUNK, CH, D)
    idx3 = node_graph_idx.astype(jnp.int32).reshape(NCHUNK, JROWS, JLEN)
    ones_h = jnp.ones((CH, CW), jnp.float32)
    z64_h = jnp.zeros((GPT, D), jnp.float32)
    z1_h = jnp.zeros((GPT, CW), jnp.float32)

    psum, pcnt = _make_sc_call()(x3, idx3, ones_h, z64_h, z1_h)

    out = pl.pallas_call(
        _tc_body,
        out_shape=jax.ShapeDtypeStruct((G, 1), jnp.float32),
    )(psum, pcnt, W, b.reshape(1, 1))
    return out
